# Initial kernel scaffold; baseline (speedup 1.0000x reference)
import jax
import jax.numpy as jnp
from jax.experimental import pallas as pl


def kernel(x):
    raise NotImplementedError("write your pallas kernel here")


# speedup vs baseline: 154.4139x; 154.4139x over previous
import jax
import jax.numpy as jnp
from jax.experimental import pallas as pl
from jax.experimental.pallas import tpu as pltpu

_B, _T, _D = 4, 4096, 2048
_R = 512
_EPS = 1e-12
_PAD = -2.0

_NA = 32
_BA = _T // _NA
_NR = 32
_IC = _T // _NR
_NC = 8
_BC = _T // _NC


def _adj_kernel(a_ref, n_ref, adj_ref):
    af = a_ref[...]
    a = af.astype(jnp.bfloat16)
    bf = jnp.concatenate([af[1:], n_ref[0]], axis=0)
    b = bf.astype(jnp.bfloat16)
    s = jax.lax.dot_general(a, b, (((1,), (1,)), ((), ())),
                            preferred_element_type=jnp.float32)
    r = jax.lax.broadcasted_iota(jnp.int32, (_BA, _BA), 0)
    c = jax.lax.broadcasted_iota(jnp.int32, (_BA, _BA), 1)
    dots = jnp.sum(jnp.where(r == c, s, 0.0), axis=1, keepdims=True)
    k = pl.program_id(0)
    gidx = k * _BA + jax.lax.broadcasted_iota(jnp.int32, (_BA, 1), 0)
    adj_ref[0] = jnp.where(gidx == _T - 1, _PAD, dots)


def _rank_kernel(ai_ref, aj_ref, rank_ref):
    gi = pl.program_id(0)
    ai = ai_ref[0]
    aj = aj_ref[...]
    jidx = jax.lax.broadcasted_iota(jnp.int32, (_IC, _T), 1)
    iidx = gi * _IC + jax.lax.broadcasted_iota(jnp.int32, (_IC, _T), 0)
    gt = aj > ai
    tie = (aj == ai) & (jidx < iidx)
    cnt = jnp.sum(jnp.where(gt | tie, 1.0, 0.0), axis=1, keepdims=True)
    rank_ref[0] = cnt.astype(jnp.int32)


def _greedy_kernel(rank_ref, wsel_ref, wrem_ref, order_ref, used_ref, sidx_ref):
    def inv(i, c):
        order_ref[rank_ref[i]] = i
        return c
    jax.lax.fori_loop(0, _T, inv, 0, unroll=8)

    def zu(i, c):
        used_ref[i] = 0
        return c
    jax.lax.fori_loop(0, _T + 8, zu, 0, unroll=8)

    def zs(i, c):
        sidx_ref[i] = 2 * _T
        return c
    jax.lax.fori_loop(0, _R, zs, 0, unroll=8)

    def cond(c):
        k, cnt = c
        return (k < _T) & (cnt < _R)

    def body(c):
        k, cnt = c
        i = order_ref[k]
        u0 = used_ref[i]
        u1 = used_ref[i + 1]
        ok = (u0 + u1) == 0
        oki = jnp.where(ok, 1, 0)
        used_ref[i] = u0 | oki
        used_ref[i + 1] = u1 | oki
        sidx_ref[cnt] = jnp.where(ok, i, 2 * _T)
        return (k + 1, cnt + oki)

    jax.lax.while_loop(cond, body, (jnp.int32(0), jnp.int32(0)))

    gidx = _IC * jax.lax.broadcasted_iota(jnp.int32, (_NR, _IC), 0) + \
        jax.lax.broadcasted_iota(jnp.int32, (_NR, _IC), 1)

    def mk(k, carry):
        ws, wr = carry
        s = sidx_ref[k]
        ws = ws + jnp.where(gidx == s, 0.5, 0.0)
        wr = wr + jnp.where(gidx == s + 1, 0.5, 0.0)
        return (ws, wr)

    z = jnp.zeros((_NR, _IC), jnp.float32)
    ws, wr = jax.lax.fori_loop(0, _R, mk, (z, z), unroll=4)
    wsel_ref[...] = ws
    wrem_ref[...] = wr


def _merge_kernel(x_ref, xl_ref, xf_ref, ws_ref, wr_ref, o_ref):
    xb = x_ref[0]
    pr = xl_ref[0, 0]
    nx = xf_ref[0, 0]
    ws = ws_ref[0]
    wr = wr_ref[0]
    xprev = jnp.concatenate([pr, xb[:-1]], axis=0)
    xnext = jnp.concatenate([xb[1:], nx], axis=0)
    o_ref[0] = (1.0 - ws - wr) * xb + ws * xnext + wr * xprev


def kernel(x):
    xs = jax.lax.stop_gradient(x)[0]
    norm = jnp.maximum(jnp.linalg.norm(xs, axis=-1, keepdims=True), _EPS)
    xn = xs / norm
    xnf = xn[::_BA].reshape(_NA, 1, _D)

    adj3 = pl.pallas_call(
        _adj_kernel,
        grid=(_NA,),
        in_specs=[
            pl.BlockSpec((_BA, _D), lambda k: (k, 0)),
            pl.BlockSpec((1, 1, _D), lambda k: (jnp.minimum(k + 1, _NA - 1), 0, 0)),
        ],
        out_specs=pl.BlockSpec((1, _BA, 1), lambda k: (k, 0, 0)),
        out_shape=jax.ShapeDtypeStruct((_NA, _BA, 1), jnp.float32),
        compiler_params=pltpu.CompilerParams(
            dimension_semantics=("arbitrary",)),
        name="tome_adj",
    )(xn, xnf)
    adj = adj3.reshape(_T)

    rank3 = pl.pallas_call(
        _rank_kernel,
        grid=(_NR,),
        in_specs=[
            pl.BlockSpec((1, _IC, 1), lambda g: (g, 0, 0)),
            pl.BlockSpec((1, _T), lambda g: (0, 0)),
        ],
        out_specs=pl.BlockSpec((1, _IC, 1), lambda g: (g, 0, 0)),
        out_shape=jax.ShapeDtypeStruct((_NR, _IC, 1), jnp.int32),
        compiler_params=pltpu.CompilerParams(
            dimension_semantics=("arbitrary",)),
        name="tome_rank",
    )(adj.reshape(_NR, _IC, 1), adj.reshape(1, _T))

    ws2, wr2 = pl.pallas_call(
        _greedy_kernel,
        in_specs=[pl.BlockSpec(memory_space=pltpu.SMEM)],
        out_shape=[
            jax.ShapeDtypeStruct((_NR, _IC), jnp.float32),
            jax.ShapeDtypeStruct((_NR, _IC), jnp.float32),
        ],
        scratch_shapes=[
            pltpu.SMEM((_T,), jnp.int32),
            pltpu.SMEM((_T + 8,), jnp.int32),
            pltpu.SMEM((_R,), jnp.int32),
        ],
        name="tome_greedy",
    )(rank3.reshape(_T))

    ws3 = ws2.reshape(_NC, _BC, 1)
    wr3 = wr2.reshape(_NC, _BC, 1)
    xl = x[:, _BC - 1::_BC].reshape(_B, _NC, 1, _D)
    xf = x[:, ::_BC].reshape(_B, _NC, 1, _D)

    out = pl.pallas_call(
        _merge_kernel,
        grid=(_B, _NC),
        in_specs=[
            pl.BlockSpec((1, _BC, _D), lambda b, k: (b, k, 0)),
            pl.BlockSpec((1, 1, 1, _D), lambda b, k: (b, jnp.maximum(k - 1, 0), 0, 0)),
            pl.BlockSpec((1, 1, 1, _D), lambda b, k: (b, jnp.minimum(k + 1, _NC - 1), 0, 0)),
            pl.BlockSpec((1, _BC, 1), lambda b, k: (k, 0, 0)),
            pl.BlockSpec((1, _BC, 1), lambda b, k: (k, 0, 0)),
        ],
        out_specs=pl.BlockSpec((1, _BC, _D), lambda b, k: (b, k, 0)),
        out_shape=jax.ShapeDtypeStruct((_B, _T, _D), jnp.float32),
        compiler_params=pltpu.CompilerParams(
            dimension_semantics=("parallel", "arbitrary")),
        name="tome_merge",
    )(x, xl, xf, ws3, wr3)
    return out
